# Initial kernel scaffold; baseline (speedup 1.0000x reference)
import jax
import jax.numpy as jnp
from jax.experimental import pallas as pl


def kernel(n_id, local_node_idx, edge_index, edge_type, node_type, emb, params):
    raise NotImplementedError("write your pallas kernel here")


# speedup vs baseline: 3.0554x; 3.0554x over previous
import functools

import jax
import jax.numpy as jnp
import numpy as np
from jax import lax
from jax.experimental import pallas as pl
from jax.experimental.pallas import tpu as pltpu

N = 10000
E = 320000
R = 5
H = 2
NEG = 0.2
NBLK = 25
BLK = 400


def _pre_body(x_ref, m_ref, wh_ref, wr_ref, wa_ref, hrow_ref, rest_ref, ai_ref):
    x = x_ref[:] * m_ref[:]
    hrow_ref[:] = jnp.dot(x, wh_ref[:], preferred_element_type=jnp.float32)
    rest_ref[:] = jnp.dot(x, wr_ref[:], preferred_element_type=jnp.float32)
    ai_ref[:] = jnp.dot(x, wa_ref[:], preferred_element_type=jnp.float32)


def _pre_call(x, mask_col, w_hrow, w_rest, w_ai):
    in_c = x.shape[1]
    return pl.pallas_call(
        _pre_body,
        grid=(NBLK,),
        in_specs=[
            pl.BlockSpec((BLK, in_c), lambda i: (i, 0)),
            pl.BlockSpec((BLK, 1), lambda i: (i, 0)),
            pl.BlockSpec((in_c, 144), lambda i: (0, 0)),
            pl.BlockSpec((in_c, 192), lambda i: (0, 0)),
            pl.BlockSpec((in_c, 16), lambda i: (0, 0)),
        ],
        out_specs=[
            pl.BlockSpec((BLK, 144), lambda i: (i, 0)),
            pl.BlockSpec((BLK, 192), lambda i: (i, 0)),
            pl.BlockSpec((BLK, 16), lambda i: (i, 0)),
        ],
        out_shape=[
            jax.ShapeDtypeStruct((N, 144), jnp.float32),
            jax.ShapeDtypeStruct((N, 192), jnp.float32),
            jax.ShapeDtypeStruct((N, 16), jnp.float32),
        ],
    )(x, mask_col, w_hrow, w_rest, w_ai)


def _post_body(zac_ref, rest_ref, wqkv_ref, wrel_ref, out_ref, *, last):
    zac = zac_ref[:]
    rest = rest_ref[:]
    self_node = rest[:, :128]
    x_self = rest[:, 128:192]
    qs, ks, vs = [], [], []
    for r in range(R):
        a = zac[:, 144 * r:144 * r + 128]
        d0 = zac[:, 144 * r + 128:144 * r + 129]
        d1 = zac[:, 144 * r + 129:144 * r + 130]
        z0 = jnp.where(d0 > 0, a[:, :64] / jnp.where(d0 > 0, d0, 1.0), 0.0)
        z1 = jnp.where(d1 > 0, a[:, 64:] / jnp.where(d1 > 0, d1, 1.0), 0.0)
        z = jnp.concatenate([z0, z1], axis=1) + self_node
        qkv = jnp.dot(z, wqkv_ref[r], preferred_element_type=jnp.float32)
        qs.append(qkv[:, :64])
        ks.append(qkv[:, 64:128])
        vs.append(qkv[:, 128:])
    acc = jnp.zeros_like(x_self)
    for r in range(R):
        g = [jnp.sum(qs[r] * ks[s], axis=1, keepdims=True) for s in range(R)]
        m = g[0]
        for s in range(1, R):
            m = jnp.maximum(m, g[s])
        e = [jnp.exp(gg - m) for gg in g]
        tot = e[0]
        for s in range(1, R):
            tot = tot + e[s]
        delta = e[0] / tot * vs[0]
        for s in range(1, R):
            delta = delta + e[s] / tot * vs[s]
        maskr = (jnp.sum(delta, axis=1, keepdims=True) != 0).astype(jnp.float32)
        acc = acc + wrel_ref[0, r] * (delta + x_self * maskr)
    if last:
        out_ref[:] = acc
    else:
        out_ref[:] = jnp.maximum(acc, 0.0)


def _post_call(zac, rest, wqkv, wrel, last):
    return pl.pallas_call(
        functools.partial(_post_body, last=last),
        grid=(NBLK,),
        in_specs=[
            pl.BlockSpec((BLK, 720), lambda i: (i, 0)),
            pl.BlockSpec((BLK, 192), lambda i: (i, 0)),
            pl.BlockSpec((R, 128, 192), lambda i: (0, 0, 0)),
            pl.BlockSpec((1, 8), lambda i: (0, 0)),
        ],
        out_specs=pl.BlockSpec((BLK, 64), lambda i: (i, 0)),
        out_shape=jax.ShapeDtypeStruct((N, 64), jnp.float32),
    )(zac, rest, wqkv, wrel)


def _prep_layer(p):
    att = p['node_att']
    C = att.shape[2] // 2
    in_c = p['lin_j'].shape[0]
    A_i = jnp.zeros((H * C, R * H), jnp.float32)
    A_j = jnp.zeros((H * C, R * H), jnp.float32)
    for r in range(R):
        for h in range(H):
            A_i = A_i.at[h * C:(h + 1) * C, r * H + h].set(att[r, h, :C])
            A_j = A_j.at[h * C:(h + 1) * C, r * H + h].set(att[r, h, C:])
    w_ai = jnp.pad(p['lin_i'] @ A_i, ((0, 0), (0, 6)))
    w_aj = p['lin_j'] @ A_j
    w_hrow = jnp.concatenate(
        [p['lin_j'], w_aj, jnp.zeros((in_c, 6), jnp.float32)], axis=1)
    w_rest = jnp.concatenate([p['W_self_node'], p['W_self']], axis=1)
    wqkv = jnp.concatenate([p['W_q'], p['W_k'], p['W_v']], axis=2)
    wrel = jnp.pad(p['W_relation'][:, 0], (0, 3))[None, :]
    return w_hrow, w_rest, w_ai, wqkv, wrel


def _edge_pass_jnp(hrow, ai, src, dst, etype):
    seg = dst * R + etype
    hr = hrow[src]
    col0 = (etype * 2)[:, None]
    ai_d = ai[dst]
    a0 = jnp.take_along_axis(ai_d, col0, axis=1)[:, 0]
    a1 = jnp.take_along_axis(ai_d, col0 + 1, axis=1)[:, 0]
    j0 = jnp.take_along_axis(hr, 128 + col0, axis=1)[:, 0]
    j1 = jnp.take_along_axis(hr, 129 + col0, axis=1)[:, 0]
    al0 = a0 + j0
    al1 = a1 + j1
    ex0 = jnp.exp(jnp.where(al0 > 0, al0, NEG * al0))
    ex1 = jnp.exp(jnp.where(al1 > 0, al1, NEG * al1))
    msg = jnp.concatenate(
        [ex0[:, None] * hr[:, :64], ex1[:, None] * hr[:, 64:128],
         ex0[:, None], ex1[:, None], jnp.zeros((E, 14), jnp.float32)], axis=1)
    zac = jax.ops.segment_sum(msg, seg, num_segments=N * R)
    return zac.reshape(N, 720)


def kernel(n_id, local_node_idx, edge_index, edge_type, node_type, emb, params):
    src = edge_index[0]
    dst = edge_index[1]
    mask_col = (node_type[n_id] == 0).astype(jnp.float32)[:, None]
    x = emb[local_node_idx[n_id]]

    for li, p in enumerate(params):
        w_hrow, w_rest, w_ai, wqkv, wrel = _prep_layer(p)
        mc = mask_col if li == 0 else jnp.ones((N, 1), jnp.float32)
        hrow, rest, ai = _pre_call(x, mc, w_hrow, w_rest, w_ai)
        zac = _edge_pass_jnp(hrow, ai, src, dst, edge_type)
        x = _post_call(zac, rest, wqkv, wrel, last=(li == 1))
    return jax.nn.log_softmax(x, axis=-1)


# speedup vs baseline: 4.1826x; 1.3689x over previous
import functools

import jax
import jax.numpy as jnp
import numpy as np
from jax import lax
from jax.experimental import pallas as pl
from jax.experimental.pallas import tpu as pltpu
from jax.experimental.pallas import tpu_sc as plsc

N = 10000
E = 320000
R = 5
H = 2
NEG = 0.2
NBLK = 25
BLK = 400

CH = 128
NSLOT = 4
NCH = 160
EP = NCH * CH
E_PAD = 16 * EP
GN = 2500
BR = 12544
TRASH = 12500
WBR = BR // 16
ZB = 16
RW = 80
NPAD = 10240


def _emb_gather_body(emb_hbm, idx_hbm, out_hbm, idx_v, rows_v, sem):
    info = plsc.get_sparse_core_info()
    nc = info.num_cores
    wid = lax.axis_index("s") * nc + lax.axis_index("c")
    bpw = NPAD // (nc * 16)
    base = wid * bpw
    pltpu.sync_copy(idx_hbm.at[pl.ds(base, bpw)], idx_v)
    for q in range(4):
        pltpu.async_copy(
            emb_hbm.at[idx_v.at[pl.ds(q * 80, 80)]],
            rows_v.at[pl.ds(q * 80, 80)], sem).wait()
    pltpu.sync_copy(rows_v, out_hbm.at[pl.ds(base, bpw)])


def _emb_gather(emb, idx):
    mesh = plsc.VectorSubcoreMesh(core_axis_name="c", subcore_axis_name="s")
    idx_p = jnp.pad(idx, (0, NPAD - N))
    fn = pl.kernel(
        _emb_gather_body,
        out_type=jax.ShapeDtypeStruct((NPAD, 128), jnp.float32),
        mesh=mesh,
        compiler_params=pltpu.CompilerParams(needs_layout_passes=False),
        scratch_types=[
            pltpu.VMEM((NPAD // 32,), jnp.int32),
            pltpu.VMEM((NPAD // 32, 128), jnp.float32),
            pltpu.SemaphoreType.DMA,
        ],
    )
    return fn(emb, idx_p)[:N]


def _edge_body(epack, hrow0, hrow1, aitab0, aitab1, out, z_sp, ebuf, rows,
               arows, gidx, dbuf, sidx, etb, mb, exb, zb, semE, semG, semGA,
               semS):
    c = lax.axis_index("c")
    s = lax.axis_index("s")
    iota = lax.iota(jnp.int32, 16)
    tile_base = s * EP

    zf = jnp.zeros((16,), jnp.float32)
    for i in range(ZB):
        for j in range(RW // 16):
            zb[i, pl.ds(j * 16, 16)] = zf

    def start_edge(slot, ch):
        pltpu.async_copy(
            epack.at[pl.ds(tile_base + ch * CH, CH)], ebuf.at[slot],
            semE.at[slot])

    def wait_edge(slot):
        pltpu.make_async_copy(
            epack.at[pl.ds(tile_base, CH)], ebuf.at[slot], semE.at[slot]).wait()

    def stage_a(slot, base):
        c0 = jnp.zeros((16,), jnp.int32)
        for v in range(8):
            sl = pl.ds(v * 16, 16)
            ri = iota + (v * 16)
            src_v = plsc.load_gather(ebuf.at[slot], [ri, c0])
            dst_v = plsc.load_gather(ebuf.at[slot], [ri, c0 + 1])
            et_v = plsc.load_gather(ebuf.at[slot], [ri, c0 + 2])
            dstl = dst_v - base
            ing = (dstl >= 0) & (dstl < GN)
            gidx[slot, sl] = src_v
            dbuf[slot, sl] = dst_v
            etb[slot, sl] = et_v
            sidx[slot, sl] = jnp.where(ing, dstl * R + et_v, TRASH)
            mb[slot, sl] = jnp.where(ing, 1.0, 0.0).astype(jnp.float32)

    def start_gathers(slot, hrow, aitab):
        pltpu.async_copy(hrow.at[gidx.at[slot]], rows.at[slot], semG.at[slot])
        pltpu.async_copy(aitab.at[dbuf.at[slot]], arows.at[slot], semGA.at[slot])

    def wait_gathers(slot, hrow, aitab):
        pltpu.make_async_copy(
            hrow.at[gidx.at[slot]], rows.at[slot], semG.at[slot]).wait()
        pltpu.make_async_copy(
            aitab.at[dbuf.at[slot]], arows.at[slot], semGA.at[slot]).wait()

    def stage_b(slot):
        for v in range(8):
            sl = pl.ds(v * 16, 16)
            ri = iota + (v * 16)
            e2 = etb[slot, sl]
            aj = plsc.load_gather(rows.at[slot], [ri, e2 + 64])
            a0 = plsc.load_gather(arows.at[slot], [ri, e2])
            m = mb[slot, sl]
            t0 = a0 + aj
            t0 = jnp.where(t0 > 0, t0, NEG * t0)
            exb[sl] = jnp.exp(t0) * m

        def edge_scale(e, _):
            esplat = jnp.zeros((16,), jnp.int32) + e
            mv = plsc.load_gather(mb.at[slot], [esplat])

            @pl.when(mv[0] > 0)
            def _():
                xv0 = plsc.load_gather(exb, [esplat])
                for kb in range(4):
                    cs = pl.ds(kb * 16, 16)
                    rows[slot, e, cs] = rows[slot, e, cs] * xv0
                rows[slot, e, pl.ds(64, 16)] = jnp.where(
                    iota == 0, xv0, 0.0)
            return _

        lax.fori_loop(0, CH, edge_scale, None)

    def start_scatter(slot):
        pltpu.async_copy(rows.at[slot], z_sp.at[sidx.at[slot]], semS.at[slot],
                         add=True)

    def wait_scatter(slot):
        pltpu.make_async_copy(rows.at[slot], z_sp.at[sidx.at[slot]],
                              semS.at[slot]).wait()

    for k in range(2):
      for h in range(2):
        hrow = hrow0 if h == 0 else hrow1
        aitab = aitab0 if h == 0 else aitab1
        b = 2 * c + k
        base = b * GN
        for w in range(WBR // ZB):
            pltpu.sync_copy(zb, z_sp.at[pl.ds(s * WBR + w * ZB, ZB)])
        plsc.subcore_barrier()

        for j in range(NSLOT):
            start_edge(j, j)

        def round_body(i, _):
            for j in range(NSLOT):
                wait_edge(j)
                stage_a(j, base)

                @pl.when(i > 0)
                def _():
                    wait_scatter(j)

                start_gathers(j, hrow, aitab)

                @pl.when(i < NCH // NSLOT - 1)
                def _():
                    start_edge(j, NSLOT * (i + 1) + j)

            for j in range(NSLOT):
                wait_gathers(j, hrow, aitab)
                stage_b(j)
                start_scatter(j)
            return _

        lax.fori_loop(0, NCH // NSLOT, round_body, None)
        for j in range(NSLOT):
            wait_scatter(j)
        plsc.subcore_barrier()
        pltpu.sync_copy(z_sp.at[pl.ds(s * WBR, WBR)],
                        out.at[b].at[h].at[pl.ds(s * WBR, WBR)])
        plsc.subcore_barrier()


def _edge_call(epack, hrow0, hrow1, aitab0, aitab1):
    mesh = plsc.VectorSubcoreMesh(core_axis_name="c", subcore_axis_name="s")
    fn = pl.kernel(
        _edge_body,
        out_type=jax.ShapeDtypeStruct((4, 2, BR, RW), jnp.float32),
        mesh=mesh,
        compiler_params=pltpu.CompilerParams(
            needs_layout_passes=False, use_tc_tiling_on_sc=False),
        scratch_types=[
            pltpu.VMEM_SHARED((BR, RW), jnp.float32),
            pltpu.VMEM((NSLOT, CH, 4), jnp.int32),
            pltpu.VMEM((NSLOT, CH, RW), jnp.float32),
            pltpu.VMEM((NSLOT, CH, 16), jnp.float32),
            pltpu.VMEM((NSLOT, CH), jnp.int32),
            pltpu.VMEM((NSLOT, CH), jnp.int32),
            pltpu.VMEM((NSLOT, CH), jnp.int32),
            pltpu.VMEM((NSLOT, CH), jnp.int32),
            pltpu.VMEM((NSLOT, CH), jnp.float32),
            pltpu.VMEM((CH,), jnp.float32),
            pltpu.VMEM((ZB, RW), jnp.float32),
            pltpu.SemaphoreType.DMA((NSLOT,)),
            pltpu.SemaphoreType.DMA((NSLOT,)),
            pltpu.SemaphoreType.DMA((NSLOT,)),
            pltpu.SemaphoreType.DMA((NSLOT,)),
        ],
    )
    return fn(epack, hrow0, hrow1, aitab0, aitab1)


def _pre_body(x_ref, m_ref, wh_ref, wr_ref, wa_ref,
              h0_ref, h1_ref, rest_ref, a0_ref, a1_ref):
    x = x_ref[:] * m_ref[:]
    y = jnp.dot(x, wh_ref[:], preferred_element_type=jnp.float32)
    h0_ref[:] = y[:, :RW]
    h1_ref[:] = y[:, RW:]
    rest_ref[:] = jnp.dot(x, wr_ref[:], preferred_element_type=jnp.float32)
    a = jnp.dot(x, wa_ref[:], preferred_element_type=jnp.float32)
    a0_ref[:] = a[:, :16]
    a1_ref[:] = a[:, 16:]


def _pre_call(x, mask_col, w_hrow, w_rest, w_ai):
    in_c = x.shape[1]
    return pl.pallas_call(
        _pre_body,
        grid=(NBLK,),
        in_specs=[
            pl.BlockSpec((BLK, in_c), lambda i: (i, 0)),
            pl.BlockSpec((BLK, 1), lambda i: (i, 0)),
            pl.BlockSpec((in_c, 2 * RW), lambda i: (0, 0)),
            pl.BlockSpec((in_c, 192), lambda i: (0, 0)),
            pl.BlockSpec((in_c, 32), lambda i: (0, 0)),
        ],
        out_specs=[
            pl.BlockSpec((BLK, RW), lambda i: (i, 0)),
            pl.BlockSpec((BLK, RW), lambda i: (i, 0)),
            pl.BlockSpec((BLK, 192), lambda i: (i, 0)),
            pl.BlockSpec((BLK, 16), lambda i: (i, 0)),
            pl.BlockSpec((BLK, 16), lambda i: (i, 0)),
        ],
        out_shape=[
            jax.ShapeDtypeStruct((N, RW), jnp.float32),
            jax.ShapeDtypeStruct((N, RW), jnp.float32),
            jax.ShapeDtypeStruct((N, 192), jnp.float32),
            jax.ShapeDtypeStruct((N, 16), jnp.float32),
            jax.ShapeDtypeStruct((N, 16), jnp.float32),
        ],
    )(x, mask_col, w_hrow, w_rest, w_ai)


def _post_body(zac0_ref, zac1_ref, rest_ref, wqkv_ref, wrel_ref, out_ref, *,
               last):
    zac0 = zac0_ref[:]
    zac1 = zac1_ref[:]
    rest = rest_ref[:]
    self_node = rest[:, :128]
    x_self = rest[:, 128:192]
    qs, ks, vs = [], [], []
    for r in range(R):
        a0 = zac0[:, RW * r:RW * r + 64]
        d0 = zac0[:, RW * r + 64:RW * r + 65]
        a1 = zac1[:, RW * r:RW * r + 64]
        d1 = zac1[:, RW * r + 64:RW * r + 65]
        z0 = jnp.where(d0 > 0, a0 / jnp.where(d0 > 0, d0, 1.0), 0.0)
        z1 = jnp.where(d1 > 0, a1 / jnp.where(d1 > 0, d1, 1.0), 0.0)
        z = jnp.concatenate([z0, z1], axis=1) + self_node
        qkv = jnp.dot(z, wqkv_ref[r], preferred_element_type=jnp.float32)
        qs.append(qkv[:, :64])
        ks.append(qkv[:, 64:128])
        vs.append(qkv[:, 128:])
    acc = jnp.zeros_like(x_self)
    for r in range(R):
        g = [jnp.sum(qs[r] * ks[s], axis=1, keepdims=True) for s in range(R)]
        m = g[0]
        for s in range(1, R):
            m = jnp.maximum(m, g[s])
        e = [jnp.exp(gg - m) for gg in g]
        tot = e[0]
        for s in range(1, R):
            tot = tot + e[s]
        delta = e[0] / tot * vs[0]
        for s in range(1, R):
            delta = delta + e[s] / tot * vs[s]
        maskr = (jnp.sum(delta, axis=1, keepdims=True) != 0).astype(jnp.float32)
        acc = acc + wrel_ref[0, r] * (delta + x_self * maskr)
    if last:
        out_ref[:] = acc
    else:
        out_ref[:] = jnp.maximum(acc, 0.0)


def _post_call(zac0, zac1, rest, wqkv, wrel, last):
    return pl.pallas_call(
        functools.partial(_post_body, last=last),
        grid=(NBLK,),
        in_specs=[
            pl.BlockSpec((BLK, R * RW), lambda i: (i, 0)),
            pl.BlockSpec((BLK, R * RW), lambda i: (i, 0)),
            pl.BlockSpec((BLK, 192), lambda i: (i, 0)),
            pl.BlockSpec((R, 128, 192), lambda i: (0, 0, 0)),
            pl.BlockSpec((1, 8), lambda i: (0, 0)),
        ],
        out_specs=pl.BlockSpec((BLK, 64), lambda i: (i, 0)),
        out_shape=jax.ShapeDtypeStruct((N, 64), jnp.float32),
    )(zac0, zac1, rest, wqkv, wrel)


def _prep_layer(p):
    att = p['node_att']
    C = att.shape[2] // 2
    in_c = p['lin_j'].shape[0]
    A_i = jnp.zeros((H * C, R * H), jnp.float32)
    A_j = jnp.zeros((H * C, R * H), jnp.float32)
    for r in range(R):
        for h in range(H):
            A_i = A_i.at[h * C:(h + 1) * C, r * H + h].set(att[r, h, :C])
            A_j = A_j.at[h * C:(h + 1) * C, r * H + h].set(att[r, h, C:])
    wai_f = p['lin_i'] @ A_i
    waj_f = p['lin_j'] @ A_j
    zpad11 = jnp.zeros((in_c, 11), jnp.float32)
    zpad5 = jnp.zeros((in_c, 11), jnp.float32)[:, :5]
    hrow_parts = []
    ai_parts = []
    for h in range(H):
        hj_h = p['lin_j'][:, h * 64:(h + 1) * 64]
        aj_h = waj_f[:, h::H]
        hrow_parts.append(jnp.concatenate([hj_h, aj_h, zpad11], axis=1))
        ai_h = wai_f[:, h::H]
        ai_parts.append(jnp.concatenate([ai_h, zpad11, zpad5[:, :0]], axis=1))
    w_hrow = jnp.concatenate(hrow_parts, axis=1)
    w_ai = jnp.concatenate(ai_parts, axis=1)
    w_rest = jnp.concatenate([p['W_self_node'], p['W_self']], axis=1)
    wqkv = jnp.concatenate([p['W_q'], p['W_k'], p['W_v']], axis=2)
    wrel = jnp.pad(p['W_relation'][:, 0], (0, 3))[None, :]
    return w_hrow, w_rest, w_ai, wqkv, wrel


def kernel(n_id, local_node_idx, edge_index, edge_type, node_type, emb, params):
    src = edge_index[0]
    dst = edge_index[1]
    mask_col = (node_type[n_id] == 0).astype(jnp.float32)[:, None]

    pad = E_PAD - E
    srcp = jnp.pad(src, (0, pad))
    dstp = jnp.pad(dst, (0, pad), constant_values=N - 1)
    etp = jnp.pad(edge_type, (0, pad), constant_values=7)
    epack = jnp.stack(
        [srcp, dstp, etp, jnp.zeros((E_PAD,), jnp.int32)], axis=1)

    x = _emb_gather(emb, local_node_idx[n_id])

    for li, p in enumerate(params):
        w_hrow, w_rest, w_ai, wqkv, wrel = _prep_layer(p)
        mc = mask_col if li == 0 else jnp.ones((N, 1), jnp.float32)
        hrow0, hrow1, rest, ai0, ai1 = _pre_call(x, mc, w_hrow, w_rest, w_ai)
        zout = _edge_call(epack, hrow0, hrow1, ai0, ai1)
        zac0 = zout[:, 0, :TRASH, :].reshape(N, R * RW)
        zac1 = zout[:, 1, :TRASH, :].reshape(N, R * RW)
        x = _post_call(zac0, zac1, rest, wqkv, wrel, last=(li == 1))
    return jax.nn.log_softmax(x, axis=-1)


# speedup vs baseline: 5.0957x; 1.2183x over previous
import functools

import jax
import jax.numpy as jnp
import numpy as np
from jax import lax
from jax.experimental import pallas as pl
from jax.experimental.pallas import tpu as pltpu
from jax.experimental.pallas import tpu_sc as plsc

N = 10000
E = 320000
R = 5
H = 2
NEG = 0.2
NBLK = 25
BLK = 400

CH = 128
NSLOT = 4
NCH = 160
EP = NCH * CH
E_PAD = 16 * EP
GN = 2500
BR = 12544
TRASH = 12500
WBR = BR // 16
ZB = 16
RW = 80
NPAD = 10240


def _emb_gather_body(emb_hbm, idx_hbm, out_hbm, idx_v, rows_v, sem):
    info = plsc.get_sparse_core_info()
    nc = info.num_cores
    wid = lax.axis_index("s") * nc + lax.axis_index("c")
    bpw = NPAD // (nc * 16)
    base = wid * bpw
    pltpu.sync_copy(idx_hbm.at[pl.ds(base, bpw)], idx_v)
    for q in range(4):
        pltpu.async_copy(
            emb_hbm.at[idx_v.at[pl.ds(q * 80, 80)]],
            rows_v.at[pl.ds(q * 80, 80)], sem).wait()
    pltpu.sync_copy(rows_v, out_hbm.at[pl.ds(base, bpw)])


def _emb_gather(emb, idx):
    mesh = plsc.VectorSubcoreMesh(core_axis_name="c", subcore_axis_name="s")
    idx_p = jnp.pad(idx, (0, NPAD - N))
    fn = pl.kernel(
        _emb_gather_body,
        out_type=jax.ShapeDtypeStruct((NPAD, 128), jnp.float32),
        mesh=mesh,
        compiler_params=pltpu.CompilerParams(needs_layout_passes=False),
        scratch_types=[
            pltpu.VMEM((NPAD // 32,), jnp.int32),
            pltpu.VMEM((NPAD // 32, 128), jnp.float32),
            pltpu.SemaphoreType.DMA,
        ],
    )
    return fn(emb, idx_p)[:N]


def _edge_body(epack, hrow0, hrow1, aitab0, aitab1, out, z_sp, ebuf, rows,
               arows, gidx, dbuf, sidx, etb, mb, exb, zb, semE, semG, semGA,
               semS):
    c = lax.axis_index("c")
    s = lax.axis_index("s")
    iota = lax.iota(jnp.int32, 16)
    tile_base = s * EP

    zf = jnp.zeros((16,), jnp.float32)
    for i in range(ZB):
        for j in range(RW // 16):
            zb[i, pl.ds(j * 16, 16)] = zf

    def start_edge(slot, ch):
        pltpu.async_copy(
            epack.at[pl.ds(tile_base + ch * CH, CH)], ebuf.at[slot],
            semE.at[slot])

    def wait_edge(slot):
        pltpu.make_async_copy(
            epack.at[pl.ds(tile_base, CH)], ebuf.at[slot], semE.at[slot]).wait()

    def stage_a(slot, base):
        c0 = jnp.zeros((16,), jnp.int32)
        for v in range(8):
            sl = pl.ds(v * 16, 16)
            ri = iota + (v * 16)
            src_v = plsc.load_gather(ebuf.at[slot], [ri, c0])
            dst_v = plsc.load_gather(ebuf.at[slot], [ri, c0 + 1])
            et_v = plsc.load_gather(ebuf.at[slot], [ri, c0 + 2])
            dstl = dst_v - base
            ing = (dstl >= 0) & (dstl < GN)
            gidx[slot, sl] = src_v
            dbuf[slot, sl] = dst_v
            etb[slot, sl] = et_v
            sidx[slot, sl] = jnp.where(ing, dstl * R + et_v, TRASH)
            mb[slot, sl] = jnp.where(ing, 1.0, 0.0).astype(jnp.float32)

    def start_gathers(slot, hrow, aitab):
        pltpu.async_copy(hrow.at[gidx.at[slot]], rows.at[slot], semG.at[slot])
        pltpu.async_copy(aitab.at[dbuf.at[slot]], arows.at[slot], semGA.at[slot])

    def wait_gathers(slot, hrow, aitab):
        pltpu.make_async_copy(
            hrow.at[gidx.at[slot]], rows.at[slot], semG.at[slot]).wait()
        pltpu.make_async_copy(
            aitab.at[dbuf.at[slot]], arows.at[slot], semGA.at[slot]).wait()

    def stage_b(slot):
        for v in range(8):
            sl = pl.ds(v * 16, 16)
            ri = iota + (v * 16)
            e2 = etb[slot, sl]
            aj = plsc.load_gather(rows.at[slot], [ri, e2 + 64])
            a0 = plsc.load_gather(arows.at[slot], [ri, e2])
            m = mb[slot, sl]
            t0 = a0 + aj
            t0 = jnp.where(t0 > 0, t0, NEG * t0)
            exb[sl] = jnp.exp(t0) * m

        def vec_scale(v, _):
            exv = exb[pl.ds(v * 16, 16)]
            den = jnp.where(iota == 0, 1.0, 0.0)
            for el in range(16):
                e = v * 16 + el
                x0 = exv[el]
                for kb in range(4):
                    cs = pl.ds(kb * 16, 16)
                    rows[slot, e, cs] = rows[slot, e, cs] * x0
                rows[slot, e, pl.ds(64, 16)] = den * x0
            return _

        lax.fori_loop(0, 8, vec_scale, None)

    def start_scatter(slot):
        pltpu.async_copy(rows.at[slot], z_sp.at[sidx.at[slot]], semS.at[slot],
                         add=True)

    def wait_scatter(slot):
        pltpu.make_async_copy(rows.at[slot], z_sp.at[sidx.at[slot]],
                              semS.at[slot]).wait()

    for h in range(2):
        hrow = hrow0 if h == 0 else hrow1
        aitab = aitab0 if h == 0 else aitab1

        def phase(k, _, hrow=hrow, aitab=aitab, h=h):
            b = 2 * c + k
            base = b * GN
            for w in range(WBR // ZB):
                pltpu.sync_copy(zb, z_sp.at[pl.ds(s * WBR + w * ZB, ZB)])
            plsc.subcore_barrier()

            for j in range(NSLOT):
                start_edge(j, j)

            def round_body(i, _):
                for j in range(NSLOT):
                    wait_edge(j)
                    stage_a(j, base)

                    @pl.when(i > 0)
                    def _():
                        wait_scatter(j)

                    start_gathers(j, hrow, aitab)

                    @pl.when(i < NCH // NSLOT - 1)
                    def _():
                        start_edge(j, NSLOT * (i + 1) + j)

                for j in range(NSLOT):
                    wait_gathers(j, hrow, aitab)
                    stage_b(j)
                    start_scatter(j)
                return _

            lax.fori_loop(0, NCH // NSLOT, round_body, None)
            for j in range(NSLOT):
                wait_scatter(j)
            plsc.subcore_barrier()
            pltpu.sync_copy(z_sp.at[pl.ds(s * WBR, WBR)],
                            out.at[b].at[h].at[pl.ds(s * WBR, WBR)])
            plsc.subcore_barrier()
            return _

        lax.fori_loop(0, 2, phase, None)


def _edge_call(epack, hrow0, hrow1, aitab0, aitab1):
    mesh = plsc.VectorSubcoreMesh(core_axis_name="c", subcore_axis_name="s")
    fn = pl.kernel(
        _edge_body,
        out_type=jax.ShapeDtypeStruct((4, 2, BR, RW), jnp.float32),
        mesh=mesh,
        compiler_params=pltpu.CompilerParams(
            needs_layout_passes=False, use_tc_tiling_on_sc=False),
        scratch_types=[
            pltpu.VMEM_SHARED((BR, RW), jnp.float32),
            pltpu.VMEM((NSLOT, CH, 4), jnp.int32),
            pltpu.VMEM((NSLOT, CH, RW), jnp.float32),
            pltpu.VMEM((NSLOT, CH, 16), jnp.float32),
            pltpu.VMEM((NSLOT, CH), jnp.int32),
            pltpu.VMEM((NSLOT, CH), jnp.int32),
            pltpu.VMEM((NSLOT, CH), jnp.int32),
            pltpu.VMEM((NSLOT, CH), jnp.int32),
            pltpu.VMEM((NSLOT, CH), jnp.float32),
            pltpu.VMEM((CH,), jnp.float32),
            pltpu.VMEM((ZB, RW), jnp.float32),
            pltpu.SemaphoreType.DMA((NSLOT,)),
            pltpu.SemaphoreType.DMA((NSLOT,)),
            pltpu.SemaphoreType.DMA((NSLOT,)),
            pltpu.SemaphoreType.DMA((NSLOT,)),
        ],
    )
    return fn(epack, hrow0, hrow1, aitab0, aitab1)


def _pre_body(x_ref, m_ref, wh_ref, wr_ref, wa_ref,
              h0_ref, h1_ref, rest_ref, a0_ref, a1_ref):
    x = x_ref[:] * m_ref[:]
    y = jnp.dot(x, wh_ref[:], preferred_element_type=jnp.float32)
    h0_ref[:] = y[:, :RW]
    h1_ref[:] = y[:, RW:]
    rest_ref[:] = jnp.dot(x, wr_ref[:], preferred_element_type=jnp.float32)
    a = jnp.dot(x, wa_ref[:], preferred_element_type=jnp.float32)
    a0_ref[:] = a[:, :16]
    a1_ref[:] = a[:, 16:]


def _pre_call(x, mask_col, w_hrow, w_rest, w_ai):
    in_c = x.shape[1]
    return pl.pallas_call(
        _pre_body,
        grid=(NBLK,),
        in_specs=[
            pl.BlockSpec((BLK, in_c), lambda i: (i, 0)),
            pl.BlockSpec((BLK, 1), lambda i: (i, 0)),
            pl.BlockSpec((in_c, 2 * RW), lambda i: (0, 0)),
            pl.BlockSpec((in_c, 192), lambda i: (0, 0)),
            pl.BlockSpec((in_c, 32), lambda i: (0, 0)),
        ],
        out_specs=[
            pl.BlockSpec((BLK, RW), lambda i: (i, 0)),
            pl.BlockSpec((BLK, RW), lambda i: (i, 0)),
            pl.BlockSpec((BLK, 192), lambda i: (i, 0)),
            pl.BlockSpec((BLK, 16), lambda i: (i, 0)),
            pl.BlockSpec((BLK, 16), lambda i: (i, 0)),
        ],
        out_shape=[
            jax.ShapeDtypeStruct((N, RW), jnp.float32),
            jax.ShapeDtypeStruct((N, RW), jnp.float32),
            jax.ShapeDtypeStruct((N, 192), jnp.float32),
            jax.ShapeDtypeStruct((N, 16), jnp.float32),
            jax.ShapeDtypeStruct((N, 16), jnp.float32),
        ],
    )(x, mask_col, w_hrow, w_rest, w_ai)


def _post_body(zac0_ref, zac1_ref, rest_ref, wqkv_ref, wrel_ref, out_ref, *,
               last):
    zac0 = zac0_ref[:]
    zac1 = zac1_ref[:]
    rest = rest_ref[:]
    self_node = rest[:, :128]
    x_self = rest[:, 128:192]
    qs, ks, vs = [], [], []
    for r in range(R):
        a0 = zac0[:, RW * r:RW * r + 64]
        d0 = zac0[:, RW * r + 64:RW * r + 65]
        a1 = zac1[:, RW * r:RW * r + 64]
        d1 = zac1[:, RW * r + 64:RW * r + 65]
        z0 = jnp.where(d0 > 0, a0 / jnp.where(d0 > 0, d0, 1.0), 0.0)
        z1 = jnp.where(d1 > 0, a1 / jnp.where(d1 > 0, d1, 1.0), 0.0)
        z = jnp.concatenate([z0, z1], axis=1) + self_node
        qkv = jnp.dot(z, wqkv_ref[r], preferred_element_type=jnp.float32)
        qs.append(qkv[:, :64])
        ks.append(qkv[:, 64:128])
        vs.append(qkv[:, 128:])
    acc = jnp.zeros_like(x_self)
    for r in range(R):
        g = [jnp.sum(qs[r] * ks[s], axis=1, keepdims=True) for s in range(R)]
        m = g[0]
        for s in range(1, R):
            m = jnp.maximum(m, g[s])
        e = [jnp.exp(gg - m) for gg in g]
        tot = e[0]
        for s in range(1, R):
            tot = tot + e[s]
        delta = e[0] / tot * vs[0]
        for s in range(1, R):
            delta = delta + e[s] / tot * vs[s]
        maskr = (jnp.sum(delta, axis=1, keepdims=True) != 0).astype(jnp.float32)
        acc = acc + wrel_ref[0, r] * (delta + x_self * maskr)
    if last:
        out_ref[:] = acc
    else:
        out_ref[:] = jnp.maximum(acc, 0.0)


def _post_call(zac0, zac1, rest, wqkv, wrel, last):
    return pl.pallas_call(
        functools.partial(_post_body, last=last),
        grid=(NBLK,),
        in_specs=[
            pl.BlockSpec((BLK, R * RW), lambda i: (i, 0)),
            pl.BlockSpec((BLK, R * RW), lambda i: (i, 0)),
            pl.BlockSpec((BLK, 192), lambda i: (i, 0)),
            pl.BlockSpec((R, 128, 192), lambda i: (0, 0, 0)),
            pl.BlockSpec((1, 8), lambda i: (0, 0)),
        ],
        out_specs=pl.BlockSpec((BLK, 64), lambda i: (i, 0)),
        out_shape=jax.ShapeDtypeStruct((N, 64), jnp.float32),
    )(zac0, zac1, rest, wqkv, wrel)


def _prep_layer(p):
    att = p['node_att']
    C = att.shape[2] // 2
    in_c = p['lin_j'].shape[0]
    A_i = jnp.zeros((H * C, R * H), jnp.float32)
    A_j = jnp.zeros((H * C, R * H), jnp.float32)
    for r in range(R):
        for h in range(H):
            A_i = A_i.at[h * C:(h + 1) * C, r * H + h].set(att[r, h, :C])
            A_j = A_j.at[h * C:(h + 1) * C, r * H + h].set(att[r, h, C:])
    wai_f = p['lin_i'] @ A_i
    waj_f = p['lin_j'] @ A_j
    zpad11 = jnp.zeros((in_c, 11), jnp.float32)
    zpad5 = jnp.zeros((in_c, 11), jnp.float32)[:, :5]
    hrow_parts = []
    ai_parts = []
    for h in range(H):
        hj_h = p['lin_j'][:, h * 64:(h + 1) * 64]
        aj_h = waj_f[:, h::H]
        hrow_parts.append(jnp.concatenate([hj_h, aj_h, zpad11], axis=1))
        ai_h = wai_f[:, h::H]
        ai_parts.append(jnp.concatenate([ai_h, zpad11, zpad5[:, :0]], axis=1))
    w_hrow = jnp.concatenate(hrow_parts, axis=1)
    w_ai = jnp.concatenate(ai_parts, axis=1)
    w_rest = jnp.concatenate([p['W_self_node'], p['W_self']], axis=1)
    wqkv = jnp.concatenate([p['W_q'], p['W_k'], p['W_v']], axis=2)
    wrel = jnp.pad(p['W_relation'][:, 0], (0, 3))[None, :]
    return w_hrow, w_rest, w_ai, wqkv, wrel


def kernel(n_id, local_node_idx, edge_index, edge_type, node_type, emb, params):
    src = edge_index[0]
    dst = edge_index[1]
    mask_col = (node_type[n_id] == 0).astype(jnp.float32)[:, None]

    pad = E_PAD - E
    srcp = jnp.pad(src, (0, pad))
    dstp = jnp.pad(dst, (0, pad), constant_values=N - 1)
    etp = jnp.pad(edge_type, (0, pad), constant_values=7)
    epack = jnp.stack(
        [srcp, dstp, etp, jnp.zeros((E_PAD,), jnp.int32)], axis=1)

    x = _emb_gather(emb, local_node_idx[n_id])

    for li, p in enumerate(params):
        w_hrow, w_rest, w_ai, wqkv, wrel = _prep_layer(p)
        mc = mask_col if li == 0 else jnp.ones((N, 1), jnp.float32)
        hrow0, hrow1, rest, ai0, ai1 = _pre_call(x, mc, w_hrow, w_rest, w_ai)
        zout = _edge_call(epack, hrow0, hrow1, ai0, ai1)
        zac0 = zout[:, 0, :TRASH, :].reshape(N, R * RW)
        zac1 = zout[:, 1, :TRASH, :].reshape(N, R * RW)
        x = _post_call(zac0, zac1, rest, wqkv, wrel, last=(li == 1))
    return jax.nn.log_softmax(x, axis=-1)


# speedup vs baseline: 8.9625x; 1.7588x over previous
import functools

import jax
import jax.numpy as jnp
import numpy as np
from jax import lax
from jax.experimental import pallas as pl
from jax.experimental.pallas import tpu as pltpu
from jax.experimental.pallas import tpu_sc as plsc

N = 10000
E = 320000
R = 5
H = 2
NEG = 0.2
NBLK = 25
BLK = 400

CH = 128
NSLOT = 4
NCH = 160
EP = NCH * CH
E_PAD = 16 * EP
GN = 2500
BR = 12544
TRASH = 12500
WBR = BR // 16
ZB = 16
CAPT = 10240
STG = 10240
RW = 80
NPAD = 10240


def _emb_gather_body(emb_hbm, idx_hbm, out_hbm, idx_v, rows_v, sem):
    info = plsc.get_sparse_core_info()
    nc = info.num_cores
    wid = lax.axis_index("s") * nc + lax.axis_index("c")
    bpw = NPAD // (nc * 16)
    base = wid * bpw
    pltpu.sync_copy(idx_hbm.at[pl.ds(base, bpw)], idx_v)
    for q in range(4):
        pltpu.async_copy(
            emb_hbm.at[idx_v.at[pl.ds(q * 80, 80)]],
            rows_v.at[pl.ds(q * 80, 80)], sem).wait()
    pltpu.sync_copy(rows_v, out_hbm.at[pl.ds(base, bpw)])


def _emb_gather(emb, idx):
    mesh = plsc.VectorSubcoreMesh(core_axis_name="c", subcore_axis_name="s")
    idx_p = jnp.pad(idx, (0, NPAD - N))
    fn = pl.kernel(
        _emb_gather_body,
        out_type=jax.ShapeDtypeStruct((NPAD, 128), jnp.float32),
        mesh=mesh,
        compiler_params=pltpu.CompilerParams(needs_layout_passes=False),
        scratch_types=[
            pltpu.VMEM((NPAD // 32,), jnp.int32),
            pltpu.VMEM((NPAD // 32, 128), jnp.float32),
            pltpu.SemaphoreType.DMA,
        ],
    )
    return fn(emb, idx_p)[:N]


def _bin_body(epack, bsrc, bdst, bet, counts, ebuf, stg, cbuf, semE):
    c = lax.axis_index("c")
    s = lax.axis_index("s")
    t = s * 2 + c
    iota = lax.iota(jnp.int32, 16)
    tile_base = t * CAPT

    zi = jnp.zeros((16,), jnp.int32)
    dumm = [zi, zi + (N - 1), zi + 7]

    def initloop(i, _):
        for f in range(3):
            stg[f, 0, pl.ds(i * 16, 16)] = dumm[f]
            stg[f, 1, pl.ds(i * 16, 16)] = dumm[f]
            stg[f, 2, pl.ds(i * 16, 16)] = dumm[f]
            stg[f, 3, pl.ds(i * 16, 16)] = dumm[f]
        return _

    lax.fori_loop(0, STG // 16, initloop, None)

    c0 = jnp.zeros((16,), jnp.int32)

    def chunk(ch, ptrs):
        pltpu.sync_copy(epack.at[pl.ds(tile_base + ch * CH, CH)], ebuf)
        new = ptrs
        for v in range(8):
            ri = iota + (v * 16)
            src_v = plsc.load_gather(ebuf, [ri, c0])
            dst_v = plsc.load_gather(ebuf, [ri, c0 + 1])
            et_v = plsc.load_gather(ebuf, [ri, c0 + 2])
            bin_v = dst_v // GN
            upd = []
            for b in range(4):
                p = new[b]
                msk = bin_v == b
                plsc.store_compressed(stg.at[0].at[b].at[pl.ds(p, 16)], src_v,
                                      mask=msk)
                plsc.store_compressed(stg.at[1].at[b].at[pl.ds(p, 16)], dst_v,
                                      mask=msk)
                plsc.store_compressed(stg.at[2].at[b].at[pl.ds(p, 16)], et_v,
                                      mask=msk)
                cntv = plsc.all_reduce_population_count(msk)
                upd.append(p + cntv[0])
            new = tuple(upd)
        return new

    ptrs = lax.fori_loop(0, CAPT // CH, chunk, (0, 0, 0, 0))

    for b in range(4):
        pltpu.sync_copy(stg.at[0].at[b].at[pl.ds(0, CAPT)], bsrc.at[b].at[t])
        pltpu.sync_copy(stg.at[1].at[b].at[pl.ds(0, CAPT)], bdst.at[b].at[t])
        pltpu.sync_copy(stg.at[2].at[b].at[pl.ds(0, CAPT)], bet.at[b].at[t])
    cv = jnp.where(iota == 0, ptrs[0],
                   jnp.where(iota == 1, ptrs[1],
                             jnp.where(iota == 2, ptrs[2],
                                       jnp.where(iota == 3, ptrs[3], 0))))
    cbuf[pl.ds(0, 16)] = cv
    pltpu.sync_copy(cbuf, counts.at[pl.ds(t * 16, 16)])


def _bin_call(epack):
    mesh = plsc.VectorSubcoreMesh(core_axis_name="c", subcore_axis_name="s")
    fn = pl.kernel(
        _bin_body,
        out_type=[
            jax.ShapeDtypeStruct((4, 32, CAPT), jnp.int32),
            jax.ShapeDtypeStruct((4, 32, CAPT), jnp.int32),
            jax.ShapeDtypeStruct((4, 32, CAPT), jnp.int32),
            jax.ShapeDtypeStruct((512,), jnp.int32),
        ],
        mesh=mesh,
        compiler_params=pltpu.CompilerParams(
            needs_layout_passes=False, use_tc_tiling_on_sc=False),
        scratch_types=[
            pltpu.VMEM((CH, 4), jnp.int32),
            pltpu.VMEM((3, 4, STG + 16), jnp.int32),
            pltpu.VMEM((16,), jnp.int32),
            pltpu.SemaphoreType.DMA,
        ],
    )
    return fn(epack)


def _edge_body(bsrc, bdst, bet, counts, hrow0, hrow1, aitab0, aitab1, out,
               z_sp, sbuf, dstb, etb, cbuf, rows, arows, sidx, mb, exb, zb,
               semE, semG, semGA, semS):
    c = lax.axis_index("c")
    s = lax.axis_index("s")
    iota = lax.iota(jnp.int32, 16)

    zf = jnp.zeros((16,), jnp.float32)
    for i in range(ZB):
        for j in range(RW // 16):
            zb[i, pl.ds(j * 16, 16)] = zf
    pltpu.sync_copy(counts, cbuf)

    def stage_a(slot, base):
        for v in range(8):
            sl = pl.ds(v * 16, 16)
            src_v = jnp.clip(sbuf[slot, sl], 0, N - 1)
            sbuf[slot, sl] = src_v
            dst_v = jnp.clip(dstb[slot, sl], 0, N - 1)
            dstb[slot, sl] = dst_v
            et_v = jnp.clip(etb[slot, sl], 0, 7)
            etb[slot, sl] = et_v
            dstl = dst_v - base
            ing = (dstl >= 0) & (dstl < GN)
            sidx[slot, sl] = jnp.where(ing, dstl * R + et_v, TRASH)
            mb[slot, sl] = jnp.where(ing, 1.0, 0.0).astype(jnp.float32)

    def stage_b(slot):
        for v in range(8):
            sl = pl.ds(v * 16, 16)
            ri = iota + (v * 16)
            e2 = etb[slot, sl]
            aj = plsc.load_gather(rows.at[slot], [ri, e2 + 64])
            a0 = plsc.load_gather(arows.at[slot], [ri, e2])
            m = mb[slot, sl]
            t0 = a0 + aj
            t0 = jnp.where(t0 > 0, t0, NEG * t0)
            exb[sl] = jnp.exp(t0) * m

        def vec_scale(v, _):
            exv = exb[pl.ds(v * 16, 16)]
            den = jnp.where(iota == 0, 1.0, 0.0)
            for el in range(16):
                e = v * 16 + el
                x0 = exv[el]
                for kb in range(4):
                    cs = pl.ds(kb * 16, 16)
                    rows[slot, e, cs] = rows[slot, e, cs] * x0
                rows[slot, e, pl.ds(64, 16)] = den * x0
            return _

        lax.fori_loop(0, 8, vec_scale, None)

    for h in range(2):
        hrow = hrow0 if h == 0 else hrow1
        aitab = aitab0 if h == 0 else aitab1

        def phase(k, _, hrow=hrow, aitab=aitab, h=h):
            b = 2 * c + k
            base = b * GN

            def start_edge(slot, bt, ch):
                off = pl.ds(ch * CH, CH)
                pltpu.async_copy(bsrc.at[b].at[bt].at[off], sbuf.at[slot],
                                 semE.at[slot])
                pltpu.async_copy(bdst.at[b].at[bt].at[off], dstb.at[slot],
                                 semE.at[slot])
                pltpu.async_copy(bet.at[b].at[bt].at[off], etb.at[slot],
                                 semE.at[slot])

            def wait_edge(slot, bt):
                off = pl.ds(0, CH)
                pltpu.make_async_copy(bsrc.at[b].at[bt].at[off], sbuf.at[slot],
                                      semE.at[slot]).wait()
                pltpu.make_async_copy(bdst.at[b].at[bt].at[off], dstb.at[slot],
                                      semE.at[slot]).wait()
                pltpu.make_async_copy(bet.at[b].at[bt].at[off], etb.at[slot],
                                      semE.at[slot]).wait()

            def start_gathers(slot):
                pltpu.async_copy(hrow.at[sbuf.at[slot]], rows.at[slot],
                                 semG.at[slot])
                pltpu.async_copy(aitab.at[dstb.at[slot]], arows.at[slot],
                                 semGA.at[slot])

            def wait_gathers(slot):
                pltpu.make_async_copy(hrow.at[sbuf.at[slot]], rows.at[slot],
                                      semG.at[slot]).wait()
                pltpu.make_async_copy(aitab.at[dstb.at[slot]], arows.at[slot],
                                      semGA.at[slot]).wait()

            def start_scatter(slot):
                pltpu.async_copy(rows.at[slot], z_sp.at[sidx.at[slot]],
                                 semS.at[slot], add=True)

            def wait_scatter(slot):
                pltpu.make_async_copy(rows.at[slot], z_sp.at[sidx.at[slot]],
                                      semS.at[slot]).wait()

            for w in range(WBR // ZB):
                pltpu.sync_copy(zb, z_sp.at[pl.ds(s * WBR + w * ZB, ZB)])
            plsc.subcore_barrier()

            def seg_loop(seg, _):
                bt = 2 * s + seg
                cntv = plsc.load_gather(cbuf, [iota * 0 + (bt * 16 + b)])
                cnt = cntv[0]

                for j in range(NSLOT):
                    @pl.when(j * CH < cnt)
                    def _(j=j):
                        start_edge(j, bt, j)

                def round_body(i, _):
                    for j in range(NSLOT):
                        ch = NSLOT * i + j
                        valid = ch * CH < cnt

                        @pl.when(valid)
                        def _(j=j):
                            wait_edge(j, bt)
                            stage_a(j, base)

                        @pl.when((i > 0) & ((NSLOT * (i - 1) + j) * CH < cnt))
                        def _(j=j):
                            wait_scatter(j)

                        @pl.when(valid)
                        def _(j=j):
                            start_gathers(j)

                        @pl.when((NSLOT * (i + 1) + j) * CH < cnt)
                        def _(j=j, i=i):
                            start_edge(j, bt, NSLOT * (i + 1) + j)

                    for j in range(NSLOT):
                        @pl.when((NSLOT * i + j) * CH < cnt)
                        def _(j=j):
                            wait_gathers(j)
                            stage_b(j)
                            start_scatter(j)
                    return _

                lax.fori_loop(0, CAPT // CH // NSLOT, round_body, None)
                last_round = CAPT // CH // NSLOT - 1
                for j in range(NSLOT):
                    @pl.when((NSLOT * last_round + j) * CH < cnt)
                    def _(j=j):
                        wait_scatter(j)
                return _

            lax.fori_loop(0, 2, seg_loop, None)
            plsc.subcore_barrier()
            pltpu.sync_copy(z_sp.at[pl.ds(s * WBR, WBR)],
                            out.at[b].at[h].at[pl.ds(s * WBR, WBR)])
            plsc.subcore_barrier()
            return _

        lax.fori_loop(0, 2, phase, None)


def _edge_call(bins, hrow0, hrow1, aitab0, aitab1):
    bsrc, bdst, bet, counts = bins
    mesh = plsc.VectorSubcoreMesh(core_axis_name="c", subcore_axis_name="s")
    fn = pl.kernel(
        _edge_body,
        out_type=jax.ShapeDtypeStruct((4, 2, BR, RW), jnp.float32),
        mesh=mesh,
        compiler_params=pltpu.CompilerParams(
            needs_layout_passes=False, use_tc_tiling_on_sc=False),
        scratch_types=[
            pltpu.VMEM_SHARED((BR, RW), jnp.float32),
            pltpu.VMEM((NSLOT, CH), jnp.int32),
            pltpu.VMEM((NSLOT, CH), jnp.int32),
            pltpu.VMEM((NSLOT, CH), jnp.int32),
            pltpu.VMEM((512,), jnp.int32),
            pltpu.VMEM((NSLOT, CH, RW), jnp.float32),
            pltpu.VMEM((NSLOT, CH, 16), jnp.float32),
            pltpu.VMEM((NSLOT, CH), jnp.int32),
            pltpu.VMEM((NSLOT, CH), jnp.float32),
            pltpu.VMEM((CH,), jnp.float32),
            pltpu.VMEM((ZB, RW), jnp.float32),
            pltpu.SemaphoreType.DMA((NSLOT,)),
            pltpu.SemaphoreType.DMA((NSLOT,)),
            pltpu.SemaphoreType.DMA((NSLOT,)),
            pltpu.SemaphoreType.DMA((NSLOT,)),
        ],
    )
    return fn(bsrc, bdst, bet, counts, hrow0, hrow1, aitab0, aitab1)


def _pre_body(x_ref, m_ref, wh_ref, wr_ref, wa_ref,
              h0_ref, h1_ref, rest_ref, a0_ref, a1_ref):
    x = x_ref[:] * m_ref[:]
    y = jnp.dot(x, wh_ref[:], preferred_element_type=jnp.float32)
    h0_ref[:] = y[:, :RW]
    h1_ref[:] = y[:, RW:]
    rest_ref[:] = jnp.dot(x, wr_ref[:], preferred_element_type=jnp.float32)
    a = jnp.dot(x, wa_ref[:], preferred_element_type=jnp.float32)
    a0_ref[:] = a[:, :16]
    a1_ref[:] = a[:, 16:]


def _pre_call(x, mask_col, w_hrow, w_rest, w_ai):
    in_c = x.shape[1]
    return pl.pallas_call(
        _pre_body,
        grid=(NBLK,),
        in_specs=[
            pl.BlockSpec((BLK, in_c), lambda i: (i, 0)),
            pl.BlockSpec((BLK, 1), lambda i: (i, 0)),
            pl.BlockSpec((in_c, 2 * RW), lambda i: (0, 0)),
            pl.BlockSpec((in_c, 192), lambda i: (0, 0)),
            pl.BlockSpec((in_c, 32), lambda i: (0, 0)),
        ],
        out_specs=[
            pl.BlockSpec((BLK, RW), lambda i: (i, 0)),
            pl.BlockSpec((BLK, RW), lambda i: (i, 0)),
            pl.BlockSpec((BLK, 192), lambda i: (i, 0)),
            pl.BlockSpec((BLK, 16), lambda i: (i, 0)),
            pl.BlockSpec((BLK, 16), lambda i: (i, 0)),
        ],
        out_shape=[
            jax.ShapeDtypeStruct((N, RW), jnp.float32),
            jax.ShapeDtypeStruct((N, RW), jnp.float32),
            jax.ShapeDtypeStruct((N, 192), jnp.float32),
            jax.ShapeDtypeStruct((N, 16), jnp.float32),
            jax.ShapeDtypeStruct((N, 16), jnp.float32),
        ],
    )(x, mask_col, w_hrow, w_rest, w_ai)


def _post_body(zac0_ref, zac1_ref, rest_ref, wqkv_ref, wrel_ref, out_ref, *,
               last):
    zac0 = zac0_ref[:]
    zac1 = zac1_ref[:]
    rest = rest_ref[:]
    self_node = rest[:, :128]
    x_self = rest[:, 128:192]
    qs, ks, vs = [], [], []
    for r in range(R):
        a0 = zac0[:, RW * r:RW * r + 64]
        d0 = zac0[:, RW * r + 64:RW * r + 65]
        a1 = zac1[:, RW * r:RW * r + 64]
        d1 = zac1[:, RW * r + 64:RW * r + 65]
        z0 = jnp.where(d0 > 0, a0 / jnp.where(d0 > 0, d0, 1.0), 0.0)
        z1 = jnp.where(d1 > 0, a1 / jnp.where(d1 > 0, d1, 1.0), 0.0)
        z = jnp.concatenate([z0, z1], axis=1) + self_node
        qkv = jnp.dot(z, wqkv_ref[r], preferred_element_type=jnp.float32)
        qs.append(qkv[:, :64])
        ks.append(qkv[:, 64:128])
        vs.append(qkv[:, 128:])
    acc = jnp.zeros_like(x_self)
    for r in range(R):
        g = [jnp.sum(qs[r] * ks[s], axis=1, keepdims=True) for s in range(R)]
        m = g[0]
        for s in range(1, R):
            m = jnp.maximum(m, g[s])
        e = [jnp.exp(gg - m) for gg in g]
        tot = e[0]
        for s in range(1, R):
            tot = tot + e[s]
        delta = e[0] / tot * vs[0]
        for s in range(1, R):
            delta = delta + e[s] / tot * vs[s]
        maskr = (jnp.sum(delta, axis=1, keepdims=True) != 0).astype(jnp.float32)
        acc = acc + wrel_ref[0, r] * (delta + x_self * maskr)
    if last:
        out_ref[:] = acc
    else:
        out_ref[:] = jnp.maximum(acc, 0.0)


def _post_call(zac0, zac1, rest, wqkv, wrel, last):
    return pl.pallas_call(
        functools.partial(_post_body, last=last),
        grid=(NBLK,),
        in_specs=[
            pl.BlockSpec((BLK, R * RW), lambda i: (i, 0)),
            pl.BlockSpec((BLK, R * RW), lambda i: (i, 0)),
            pl.BlockSpec((BLK, 192), lambda i: (i, 0)),
            pl.BlockSpec((R, 128, 192), lambda i: (0, 0, 0)),
            pl.BlockSpec((1, 8), lambda i: (0, 0)),
        ],
        out_specs=pl.BlockSpec((BLK, 64), lambda i: (i, 0)),
        out_shape=jax.ShapeDtypeStruct((N, 64), jnp.float32),
    )(zac0, zac1, rest, wqkv, wrel)


def _prep_layer(p):
    att = p['node_att']
    C = att.shape[2] // 2
    in_c = p['lin_j'].shape[0]
    A_i = jnp.zeros((H * C, R * H), jnp.float32)
    A_j = jnp.zeros((H * C, R * H), jnp.float32)
    for r in range(R):
        for h in range(H):
            A_i = A_i.at[h * C:(h + 1) * C, r * H + h].set(att[r, h, :C])
            A_j = A_j.at[h * C:(h + 1) * C, r * H + h].set(att[r, h, C:])
    wai_f = p['lin_i'] @ A_i
    waj_f = p['lin_j'] @ A_j
    zpad11 = jnp.zeros((in_c, 11), jnp.float32)
    zpad5 = jnp.zeros((in_c, 11), jnp.float32)[:, :5]
    hrow_parts = []
    ai_parts = []
    for h in range(H):
        hj_h = p['lin_j'][:, h * 64:(h + 1) * 64]
        aj_h = waj_f[:, h::H]
        hrow_parts.append(jnp.concatenate([hj_h, aj_h, zpad11], axis=1))
        ai_h = wai_f[:, h::H]
        ai_parts.append(jnp.concatenate([ai_h, zpad11, zpad5[:, :0]], axis=1))
    w_hrow = jnp.concatenate(hrow_parts, axis=1)
    w_ai = jnp.concatenate(ai_parts, axis=1)
    w_rest = jnp.concatenate([p['W_self_node'], p['W_self']], axis=1)
    wqkv = jnp.concatenate([p['W_q'], p['W_k'], p['W_v']], axis=2)
    wrel = jnp.pad(p['W_relation'][:, 0], (0, 3))[None, :]
    return w_hrow, w_rest, w_ai, wqkv, wrel


def kernel(n_id, local_node_idx, edge_index, edge_type, node_type, emb, params):
    src = edge_index[0]
    dst = edge_index[1]
    mask_col = (node_type[n_id] == 0).astype(jnp.float32)[:, None]

    pad = E_PAD - E
    srcp = jnp.pad(src, (0, pad))
    dstp = jnp.pad(dst, (0, pad), constant_values=N - 1)
    etp = jnp.pad(edge_type, (0, pad), constant_values=7)
    epack = jnp.stack(
        [srcp, dstp, etp, jnp.zeros((E_PAD,), jnp.int32)], axis=1)

    x = _emb_gather(emb, local_node_idx[n_id])
    bins = _bin_call(epack)

    for li, p in enumerate(params):
        w_hrow, w_rest, w_ai, wqkv, wrel = _prep_layer(p)
        mc = mask_col if li == 0 else jnp.ones((N, 1), jnp.float32)
        hrow0, hrow1, rest, ai0, ai1 = _pre_call(x, mc, w_hrow, w_rest, w_ai)
        zout = _edge_call(bins, hrow0, hrow1, ai0, ai1)
        zac0 = zout[:, 0, :TRASH, :].reshape(N, R * RW)
        zac1 = zout[:, 1, :TRASH, :].reshape(N, R * RW)
        x = _post_call(zac0, zac1, rest, wqkv, wrel, last=(li == 1))
    return jax.nn.log_softmax(x, axis=-1)


# speedup vs baseline: 9.0578x; 1.0106x over previous
import functools

import jax
import jax.numpy as jnp
import numpy as np
from jax import lax
from jax.experimental import pallas as pl
from jax.experimental.pallas import tpu as pltpu
from jax.experimental.pallas import tpu_sc as plsc

N = 10000
E = 320000
R = 5
H = 2
NEG = 0.2
NBLK = 25
BLK = 400

CH = 128
NSLOT = 4
NCH = 160
EP = NCH * CH
E_PAD = 16 * EP
GN = 2500
BR = 12544
TRASH = 12500
WBR = BR // 16
ZB = 16
CAPT = 10240
STG = 10240
RW = 80
NPAD = 10240


def _emb_gather_body(emb_hbm, idx_hbm, out_hbm, idx_v, rows_v, sem):
    info = plsc.get_sparse_core_info()
    nc = info.num_cores
    wid = lax.axis_index("s") * nc + lax.axis_index("c")
    bpw = NPAD // (nc * 16)
    base = wid * bpw
    pltpu.sync_copy(idx_hbm.at[pl.ds(base, bpw)], idx_v)
    for q in range(4):
        pltpu.async_copy(
            emb_hbm.at[idx_v.at[pl.ds(q * 80, 80)]],
            rows_v.at[pl.ds(q * 80, 80)], sem).wait()
    pltpu.sync_copy(rows_v, out_hbm.at[pl.ds(base, bpw)])


def _emb_gather(emb, idx):
    mesh = plsc.VectorSubcoreMesh(core_axis_name="c", subcore_axis_name="s")
    idx_p = jnp.pad(idx, (0, NPAD - N))
    fn = pl.kernel(
        _emb_gather_body,
        out_type=jax.ShapeDtypeStruct((NPAD, 128), jnp.float32),
        mesh=mesh,
        compiler_params=pltpu.CompilerParams(needs_layout_passes=False),
        scratch_types=[
            pltpu.VMEM((NPAD // 32,), jnp.int32),
            pltpu.VMEM((NPAD // 32, 128), jnp.float32),
            pltpu.SemaphoreType.DMA,
        ],
    )
    return fn(emb, idx_p)[:N]


def _bin_body(epack, bsrc, bdst, bet, counts, ebuf, stg, cbuf, semE):
    c = lax.axis_index("c")
    s = lax.axis_index("s")
    t = s * 2 + c
    iota = lax.iota(jnp.int32, 16)
    tile_base = t * CAPT

    zi = jnp.zeros((16,), jnp.int32)
    dumm = [zi, zi + (N - 1), zi + 7]

    def initloop(i, _):
        for f in range(3):
            stg[f, 0, pl.ds(i * 16, 16)] = dumm[f]
            stg[f, 1, pl.ds(i * 16, 16)] = dumm[f]
            stg[f, 2, pl.ds(i * 16, 16)] = dumm[f]
            stg[f, 3, pl.ds(i * 16, 16)] = dumm[f]
        return _

    lax.fori_loop(0, STG // 16, initloop, None)

    c0 = jnp.zeros((16,), jnp.int32)

    def chunk(ch, ptrs):
        pltpu.sync_copy(epack.at[pl.ds(tile_base + ch * CH, CH)], ebuf)
        new = ptrs
        for v in range(8):
            ri = iota + (v * 16)
            src_v = plsc.load_gather(ebuf, [ri, c0])
            dst_v = plsc.load_gather(ebuf, [ri, c0 + 1])
            et_v = plsc.load_gather(ebuf, [ri, c0 + 2])
            bin_v = dst_v // GN
            upd = []
            for b in range(4):
                p = new[b]
                msk = bin_v == b
                plsc.store_compressed(stg.at[0].at[b].at[pl.ds(p, 16)], src_v,
                                      mask=msk)
                plsc.store_compressed(stg.at[1].at[b].at[pl.ds(p, 16)], dst_v,
                                      mask=msk)
                plsc.store_compressed(stg.at[2].at[b].at[pl.ds(p, 16)], et_v,
                                      mask=msk)
                cntv = plsc.all_reduce_population_count(msk)
                upd.append(p + cntv[0])
            new = tuple(upd)
        return new

    ptrs = lax.fori_loop(0, CAPT // CH, chunk, (0, 0, 0, 0))

    for b in range(4):
        pltpu.sync_copy(stg.at[0].at[b].at[pl.ds(0, CAPT)], bsrc.at[b].at[t])
        pltpu.sync_copy(stg.at[1].at[b].at[pl.ds(0, CAPT)], bdst.at[b].at[t])
        pltpu.sync_copy(stg.at[2].at[b].at[pl.ds(0, CAPT)], bet.at[b].at[t])
    cv = jnp.where(iota == 0, ptrs[0],
                   jnp.where(iota == 1, ptrs[1],
                             jnp.where(iota == 2, ptrs[2],
                                       jnp.where(iota == 3, ptrs[3], 0))))
    cbuf[pl.ds(0, 16)] = cv
    pltpu.sync_copy(cbuf, counts.at[pl.ds(t * 16, 16)])


def _bin_call(epack):
    mesh = plsc.VectorSubcoreMesh(core_axis_name="c", subcore_axis_name="s")
    fn = pl.kernel(
        _bin_body,
        out_type=[
            jax.ShapeDtypeStruct((4, 32, CAPT), jnp.int32),
            jax.ShapeDtypeStruct((4, 32, CAPT), jnp.int32),
            jax.ShapeDtypeStruct((4, 32, CAPT), jnp.int32),
            jax.ShapeDtypeStruct((512,), jnp.int32),
        ],
        mesh=mesh,
        compiler_params=pltpu.CompilerParams(
            needs_layout_passes=False, use_tc_tiling_on_sc=False),
        scratch_types=[
            pltpu.VMEM((CH, 4), jnp.int32),
            pltpu.VMEM((3, 4, STG + 16), jnp.int32),
            pltpu.VMEM((16,), jnp.int32),
            pltpu.SemaphoreType.DMA,
        ],
    )
    return fn(epack)


def _edge_body(bsrc, bdst, bet, counts, hrow0, hrow1, aitab0, aitab1, out,
               z_sp, sbuf, dstb, etb, cbuf, rows, aitb, aib, sidx, mb, exb,
               zb, semE, semG, semS):
    c = lax.axis_index("c")
    s = lax.axis_index("s")
    iota = lax.iota(jnp.int32, 16)

    zf = jnp.zeros((16,), jnp.float32)
    for i in range(ZB):
        for j in range(RW // 16):
            zb[i, pl.ds(j * 16, 16)] = zf
    pltpu.sync_copy(counts, cbuf)

    def stage_a(slot, base):
        for v in range(8):
            sl = pl.ds(v * 16, 16)
            src_v = jnp.clip(sbuf[slot, sl], 0, N - 1)
            sbuf[slot, sl] = src_v
            dst_v = jnp.clip(dstb[slot, sl], 0, N - 1)
            dstb[slot, sl] = dst_v
            et_v = jnp.clip(etb[slot, sl], 0, 7)
            etb[slot, sl] = et_v
            dstl = dst_v - base
            ing = (dstl >= 0) & (dstl < GN)
            aib[slot, sl] = jnp.clip(dstl, 0, GN + 3)
            sidx[slot, sl] = jnp.where(ing, dstl * R + et_v, TRASH)
            mb[slot, sl] = jnp.where(ing, 1.0, 0.0).astype(jnp.float32)

    def stage_b(slot):
        for v in range(8):
            sl = pl.ds(v * 16, 16)
            ri = iota + (v * 16)
            e2 = etb[slot, sl]
            aj = plsc.load_gather(rows.at[slot], [ri, e2 + 64])
            a0 = plsc.load_gather(aitb, [aib[slot, sl], e2])
            m = mb[slot, sl]
            t0 = a0 + aj
            t0 = jnp.where(t0 > 0, t0, NEG * t0)
            exb[sl] = jnp.exp(t0) * m

        def vec_scale(v, _):
            exv = exb[pl.ds(v * 16, 16)]
            den = jnp.where(iota == 0, 1.0, 0.0)
            for el in range(16):
                e = v * 16 + el
                x0 = exv[el]
                for kb in range(4):
                    cs = pl.ds(kb * 16, 16)
                    rows[slot, e, cs] = rows[slot, e, cs] * x0
                rows[slot, e, pl.ds(64, 16)] = den * x0
            return _

        lax.fori_loop(0, 8, vec_scale, None)

    for h in range(2):
        hrow = hrow0 if h == 0 else hrow1
        aitab = aitab0 if h == 0 else aitab1

        def phase(k, _, hrow=hrow, aitab=aitab, h=h):
            b = 2 * c + k
            base = b * GN

            def start_edge(slot, bt, ch):
                off = pl.ds(ch * CH, CH)
                pltpu.async_copy(bsrc.at[b].at[bt].at[off], sbuf.at[slot],
                                 semE.at[slot])
                pltpu.async_copy(bdst.at[b].at[bt].at[off], dstb.at[slot],
                                 semE.at[slot])
                pltpu.async_copy(bet.at[b].at[bt].at[off], etb.at[slot],
                                 semE.at[slot])

            def wait_edge(slot, bt):
                off = pl.ds(0, CH)
                pltpu.make_async_copy(bsrc.at[b].at[bt].at[off], sbuf.at[slot],
                                      semE.at[slot]).wait()
                pltpu.make_async_copy(bdst.at[b].at[bt].at[off], dstb.at[slot],
                                      semE.at[slot]).wait()
                pltpu.make_async_copy(bet.at[b].at[bt].at[off], etb.at[slot],
                                      semE.at[slot]).wait()

            def start_gathers(slot):
                pltpu.async_copy(hrow.at[sbuf.at[slot]], rows.at[slot],
                                 semG.at[slot])

            def wait_gathers(slot):
                pltpu.make_async_copy(hrow.at[sbuf.at[slot]], rows.at[slot],
                                      semG.at[slot]).wait()

            def start_scatter(slot):
                pltpu.async_copy(rows.at[slot], z_sp.at[sidx.at[slot]],
                                 semS.at[slot], add=True)

            def wait_scatter(slot):
                pltpu.make_async_copy(rows.at[slot], z_sp.at[sidx.at[slot]],
                                      semS.at[slot]).wait()

            for w in range(WBR // ZB):
                pltpu.sync_copy(zb, z_sp.at[pl.ds(s * WBR + w * ZB, ZB)])
            pltpu.sync_copy(aitab.at[pl.ds(base, GN + 4)], aitb)
            plsc.subcore_barrier()

            def seg_loop(seg, _):
                bt = 2 * s + seg
                cntv = plsc.load_gather(cbuf, [iota * 0 + (bt * 16 + b)])
                cnt = cntv[0]

                for j in range(NSLOT):
                    @pl.when(j * CH < cnt)
                    def _(j=j):
                        start_edge(j, bt, j)

                def round_body(i, _):
                    for j in range(NSLOT):
                        ch = NSLOT * i + j
                        valid = ch * CH < cnt

                        @pl.when(valid)
                        def _(j=j):
                            wait_edge(j, bt)
                            stage_a(j, base)

                        @pl.when((i > 0) & ((NSLOT * (i - 1) + j) * CH < cnt))
                        def _(j=j):
                            wait_scatter(j)

                        @pl.when(valid)
                        def _(j=j):
                            start_gathers(j)

                        @pl.when((NSLOT * (i + 1) + j) * CH < cnt)
                        def _(j=j, i=i):
                            start_edge(j, bt, NSLOT * (i + 1) + j)

                    for j in range(NSLOT):
                        @pl.when((NSLOT * i + j) * CH < cnt)
                        def _(j=j):
                            wait_gathers(j)
                            stage_b(j)
                            start_scatter(j)
                    return _

                lax.fori_loop(0, CAPT // CH // NSLOT, round_body, None)
                last_round = CAPT // CH // NSLOT - 1
                for j in range(NSLOT):
                    @pl.when((NSLOT * last_round + j) * CH < cnt)
                    def _(j=j):
                        wait_scatter(j)
                return _

            lax.fori_loop(0, 2, seg_loop, None)
            plsc.subcore_barrier()
            pltpu.sync_copy(z_sp.at[pl.ds(s * WBR, WBR)],
                            out.at[b].at[h].at[pl.ds(s * WBR, WBR)])
            plsc.subcore_barrier()
            return _

        lax.fori_loop(0, 2, phase, None)


def _edge_call(bins, hrow0, hrow1, aitab0, aitab1):
    bsrc, bdst, bet, counts = bins
    mesh = plsc.VectorSubcoreMesh(core_axis_name="c", subcore_axis_name="s")
    fn = pl.kernel(
        _edge_body,
        out_type=jax.ShapeDtypeStruct((4, 2, BR, RW), jnp.float32),
        mesh=mesh,
        compiler_params=pltpu.CompilerParams(
            needs_layout_passes=False, use_tc_tiling_on_sc=False),
        scratch_types=[
            pltpu.VMEM_SHARED((BR, RW), jnp.float32),
            pltpu.VMEM((NSLOT, CH), jnp.int32),
            pltpu.VMEM((NSLOT, CH), jnp.int32),
            pltpu.VMEM((NSLOT, CH), jnp.int32),
            pltpu.VMEM((512,), jnp.int32),
            pltpu.VMEM((NSLOT, CH, RW), jnp.float32),
            pltpu.VMEM((GN + 4, 8), jnp.float32),
            pltpu.VMEM((NSLOT, CH), jnp.int32),
            pltpu.VMEM((NSLOT, CH), jnp.int32),
            pltpu.VMEM((NSLOT, CH), jnp.float32),
            pltpu.VMEM((CH,), jnp.float32),
            pltpu.VMEM((ZB, RW), jnp.float32),
            pltpu.SemaphoreType.DMA((NSLOT,)),
            pltpu.SemaphoreType.DMA((NSLOT,)),
            pltpu.SemaphoreType.DMA((NSLOT,)),
        ],
    )
    return fn(bsrc, bdst, bet, counts, hrow0, hrow1, aitab0, aitab1)


def _pre_body(x_ref, m_ref, wh_ref, wr_ref, wa_ref,
              h0_ref, h1_ref, rest_ref, a0_ref, a1_ref):
    x = x_ref[:] * m_ref[:]
    y = jnp.dot(x, wh_ref[:], preferred_element_type=jnp.float32)
    h0_ref[:] = y[:, :RW]
    h1_ref[:] = y[:, RW:]
    rest_ref[:] = jnp.dot(x, wr_ref[:], preferred_element_type=jnp.float32)
    a = jnp.dot(x, wa_ref[:], preferred_element_type=jnp.float32)
    a0_ref[:] = a[:, :8]
    a1_ref[:] = a[:, 8:]


def _pre_call(x, mask_col, w_hrow, w_rest, w_ai):
    in_c = x.shape[1]
    return pl.pallas_call(
        _pre_body,
        grid=(NBLK,),
        in_specs=[
            pl.BlockSpec((BLK, in_c), lambda i: (i, 0)),
            pl.BlockSpec((BLK, 1), lambda i: (i, 0)),
            pl.BlockSpec((in_c, 2 * RW), lambda i: (0, 0)),
            pl.BlockSpec((in_c, 192), lambda i: (0, 0)),
            pl.BlockSpec((in_c, 16), lambda i: (0, 0)),
        ],
        out_specs=[
            pl.BlockSpec((BLK, RW), lambda i: (i, 0)),
            pl.BlockSpec((BLK, RW), lambda i: (i, 0)),
            pl.BlockSpec((BLK, 192), lambda i: (i, 0)),
            pl.BlockSpec((BLK, 8), lambda i: (i, 0)),
            pl.BlockSpec((BLK, 8), lambda i: (i, 0)),
        ],
        out_shape=[
            jax.ShapeDtypeStruct((N, RW), jnp.float32),
            jax.ShapeDtypeStruct((N, RW), jnp.float32),
            jax.ShapeDtypeStruct((N, 192), jnp.float32),
            jax.ShapeDtypeStruct((N, 8), jnp.float32),
            jax.ShapeDtypeStruct((N, 8), jnp.float32),
        ],
    )(x, mask_col, w_hrow, w_rest, w_ai)


def _post_body(zac0_ref, zac1_ref, rest_ref, wqkv_ref, wrel_ref, out_ref, *,
               last):
    zac0 = zac0_ref[:]
    zac1 = zac1_ref[:]
    rest = rest_ref[:]
    self_node = rest[:, :128]
    x_self = rest[:, 128:192]
    qs, ks, vs = [], [], []
    for r in range(R):
        a0 = zac0[:, RW * r:RW * r + 64]
        d0 = zac0[:, RW * r + 64:RW * r + 65]
        a1 = zac1[:, RW * r:RW * r + 64]
        d1 = zac1[:, RW * r + 64:RW * r + 65]
        z0 = jnp.where(d0 > 0, a0 / jnp.where(d0 > 0, d0, 1.0), 0.0)
        z1 = jnp.where(d1 > 0, a1 / jnp.where(d1 > 0, d1, 1.0), 0.0)
        z = jnp.concatenate([z0, z1], axis=1) + self_node
        qkv = jnp.dot(z, wqkv_ref[r], preferred_element_type=jnp.float32)
        qs.append(qkv[:, :64])
        ks.append(qkv[:, 64:128])
        vs.append(qkv[:, 128:])
    acc = jnp.zeros_like(x_self)
    for r in range(R):
        g = [jnp.sum(qs[r] * ks[s], axis=1, keepdims=True) for s in range(R)]
        m = g[0]
        for s in range(1, R):
            m = jnp.maximum(m, g[s])
        e = [jnp.exp(gg - m) for gg in g]
        tot = e[0]
        for s in range(1, R):
            tot = tot + e[s]
        delta = e[0] / tot * vs[0]
        for s in range(1, R):
            delta = delta + e[s] / tot * vs[s]
        maskr = (jnp.sum(delta, axis=1, keepdims=True) != 0).astype(jnp.float32)
        acc = acc + wrel_ref[0, r] * (delta + x_self * maskr)
    if last:
        out_ref[:] = acc
    else:
        out_ref[:] = jnp.maximum(acc, 0.0)


def _post_call(zac0, zac1, rest, wqkv, wrel, last):
    return pl.pallas_call(
        functools.partial(_post_body, last=last),
        grid=(NBLK,),
        in_specs=[
            pl.BlockSpec((BLK, R * RW), lambda i: (i, 0)),
            pl.BlockSpec((BLK, R * RW), lambda i: (i, 0)),
            pl.BlockSpec((BLK, 192), lambda i: (i, 0)),
            pl.BlockSpec((R, 128, 192), lambda i: (0, 0, 0)),
            pl.BlockSpec((1, 8), lambda i: (0, 0)),
        ],
        out_specs=pl.BlockSpec((BLK, 64), lambda i: (i, 0)),
        out_shape=jax.ShapeDtypeStruct((N, 64), jnp.float32),
    )(zac0, zac1, rest, wqkv, wrel)


def _prep_layer(p):
    att = p['node_att']
    C = att.shape[2] // 2
    in_c = p['lin_j'].shape[0]
    A_i = jnp.zeros((H * C, R * H), jnp.float32)
    A_j = jnp.zeros((H * C, R * H), jnp.float32)
    for r in range(R):
        for h in range(H):
            A_i = A_i.at[h * C:(h + 1) * C, r * H + h].set(att[r, h, :C])
            A_j = A_j.at[h * C:(h + 1) * C, r * H + h].set(att[r, h, C:])
    wai_f = p['lin_i'] @ A_i
    waj_f = p['lin_j'] @ A_j
    zpad11 = jnp.zeros((in_c, 11), jnp.float32)
    zpad5 = jnp.zeros((in_c, 11), jnp.float32)[:, :5]
    hrow_parts = []
    ai_parts = []
    for h in range(H):
        hj_h = p['lin_j'][:, h * 64:(h + 1) * 64]
        aj_h = waj_f[:, h::H]
        hrow_parts.append(jnp.concatenate([hj_h, aj_h, zpad11], axis=1))
        ai_h = wai_f[:, h::H]
        ai_parts.append(jnp.concatenate([ai_h, zpad5[:, :3]], axis=1))
    w_hrow = jnp.concatenate(hrow_parts, axis=1)
    w_ai = jnp.concatenate(ai_parts, axis=1)
    w_rest = jnp.concatenate([p['W_self_node'], p['W_self']], axis=1)
    wqkv = jnp.concatenate([p['W_q'], p['W_k'], p['W_v']], axis=2)
    wrel = jnp.pad(p['W_relation'][:, 0], (0, 3))[None, :]
    return w_hrow, w_rest, w_ai, wqkv, wrel


def kernel(n_id, local_node_idx, edge_index, edge_type, node_type, emb, params):
    src = edge_index[0]
    dst = edge_index[1]
    mask_col = (node_type[n_id] == 0).astype(jnp.float32)[:, None]

    pad = E_PAD - E
    srcp = jnp.pad(src, (0, pad))
    dstp = jnp.pad(dst, (0, pad), constant_values=N - 1)
    etp = jnp.pad(edge_type, (0, pad), constant_values=7)
    epack = jnp.stack(
        [srcp, dstp, etp, jnp.zeros((E_PAD,), jnp.int32)], axis=1)

    x = _emb_gather(emb, local_node_idx[n_id])
    bins = _bin_call(epack)

    for li, p in enumerate(params):
        w_hrow, w_rest, w_ai, wqkv, wrel = _prep_layer(p)
        mc = mask_col if li == 0 else jnp.ones((N, 1), jnp.float32)
        hrow0, hrow1, rest, ai0, ai1 = _pre_call(x, mc, w_hrow, w_rest, w_ai)
        ai0 = jnp.pad(ai0, ((0, 16), (0, 0)))
        ai1 = jnp.pad(ai1, ((0, 16), (0, 0)))
        zout = _edge_call(bins, hrow0, hrow1, ai0, ai1)
        zac0 = zout[:, 0, :TRASH, :].reshape(N, R * RW)
        zac1 = zout[:, 1, :TRASH, :].reshape(N, R * RW)
        x = _post_call(zac0, zac1, rest, wqkv, wrel, last=(li == 1))
    return jax.nn.log_softmax(x, axis=-1)
